# Initial kernel scaffold; baseline (speedup 1.0000x reference)
import jax
import jax.numpy as jnp
from jax.experimental import pallas as pl


def kernel(x, edge_index, W1, b1, W2, b2):
    raise NotImplementedError("write your pallas kernel here")


# speedup vs baseline: 10.6637x; 10.6637x over previous
import functools

import jax
import jax.numpy as jnp
from jax import lax
from jax.experimental import pallas as pl
from jax.experimental.pallas import tpu as pltpu
from jax.experimental.pallas import tpu_sc as plsc

N = 10000
E = 320000
D = 128

NC = 2
NS = 16
NW = NC * NS

NP = 10240
ROWS_PER_TILE = NP // NS

CHUNK = 128
CPW = 79
EP = NW * CPW * CHUNK

EW = E // NW

R = 1024
GRID = NP // R


@functools.cache
def _sc_kernels():
    mesh = plsc.VectorSubcoreMesh(
        core_axis_name="c", subcore_axis_name="s", num_cores=NC, num_subcores=NS
    )
    deg_k = _make_deg_kernel(mesh)
    agg_k = _make_agg_kernel(mesh)
    return deg_k, agg_k


DW = 16


def _make_deg_kernel(mesh):
    @functools.partial(
        pl.kernel,
        out_type=jax.ShapeDtypeStruct((NC, NP, DW), jnp.float32),
        mesh=mesh,
        scratch_types=[
            pltpu.VMEM((CHUNK,), jnp.int32),
            pltpu.VMEM((CHUNK, DW), jnp.float32),
            pltpu.VMEM((64, DW), jnp.float32),
            pltpu.VMEM_SHARED((NP, DW), jnp.float32),
        ],
    )
    def _deg_kernel(dst_hbm, out_hbm, didx, onesb, zbuf, acc):
        cid = lax.axis_index("c")
        sid = lax.axis_index("s")
        wid = sid * NC + cid

        zeros = jnp.zeros((16,), jnp.float32)
        ones = jnp.ones((16,), jnp.float32)

        def fbody(i, _):
            r = i % 64
            zbuf[r, pl.ds(0, 16)] = zeros
            onesb[i, pl.ds(0, 16)] = ones
            return 0

        lax.fori_loop(0, CHUNK, fbody, 0)
        for t in range(ROWS_PER_TILE // 64):
            pltpu.sync_copy(zbuf, acc.at[pl.ds(sid * ROWS_PER_TILE + t * 64, 64)])
        plsc.subcore_barrier()

        def body(k, _):
            base = (wid * CPW + k) * CHUNK
            pltpu.sync_copy(dst_hbm.at[pl.ds(base, CHUNK)], didx)
            pltpu.sync_copy(onesb, acc.at[didx], add=True)
            return 0

        lax.fori_loop(0, CPW, body, 0)
        plsc.subcore_barrier()

        pltpu.sync_copy(
            acc.at[pl.ds(sid * ROWS_PER_TILE, ROWS_PER_TILE)],
            out_hbm.at[cid, pl.ds(sid * ROWS_PER_TILE, ROWS_PER_TILE)],
        )

    return _deg_kernel


def _make_agg_kernel(mesh):
    @functools.partial(
        pl.kernel,
        out_type=jax.ShapeDtypeStruct((NC, NP, D), jnp.float32),
        mesh=mesh,
        scratch_types=[
            pltpu.VMEM((CHUNK,), jnp.int32),
            pltpu.VMEM((CHUNK,), jnp.int32),
            pltpu.VMEM((CHUNK, D), jnp.float32),
            pltpu.VMEM((64, D), jnp.float32),
            pltpu.VMEM_SHARED((NP, D), jnp.float32),
            pltpu.SemaphoreType.DMA,
        ],
    )
    def _agg_kernel(g_hbm, src_hbm, dst_hbm, out_hbm, sidx, didx, rows, zbuf, acc, sem):
        cid = lax.axis_index("c")
        sid = lax.axis_index("s")
        wid = sid * NC + cid

        zeros = jnp.zeros((16,), jnp.float32)

        def zbody(i, _):
            r = i // 8
            j = i % 8
            zbuf[r, pl.ds(j * 16, 16)] = zeros
            return 0

        lax.fori_loop(0, 64 * 8, zbody, 0)
        for t in range(ROWS_PER_TILE // 64):
            pltpu.sync_copy(zbuf, acc.at[pl.ds(sid * ROWS_PER_TILE + t * 64, 64)])
        plsc.subcore_barrier()

        def body(k, _):
            base = (wid * CPW + k) * CHUNK
            pltpu.sync_copy(src_hbm.at[pl.ds(base, CHUNK)], sidx)
            pltpu.sync_copy(dst_hbm.at[pl.ds(base, CHUNK)], didx)
            pltpu.async_copy(g_hbm.at[sidx], rows, sem).wait()
            pltpu.sync_copy(rows, acc.at[didx], add=True)
            return 0

        lax.fori_loop(0, CPW, body, 0)
        plsc.subcore_barrier()

        pltpu.sync_copy(
            acc.at[pl.ds(sid * ROWS_PER_TILE, ROWS_PER_TILE)],
            out_hbm.at[cid, pl.ds(sid * ROWS_PER_TILE, ROWS_PER_TILE)],
        )

    return _agg_kernel


def _prep_body(hist_ref, x_ref, w_ref, g_ref, dinv_ref):
    deg = jnp.sum(hist_ref[...], axis=(0, 2)) + 1.0
    di = lax.rsqrt(deg)
    h = jnp.dot(x_ref[...], w_ref[...], preferred_element_type=jnp.float32)
    g_ref[...] = di[:, None] * h
    dinv_ref[...] = di[:, None]


def _mid_body(p_ref, g1_ref, dinv_ref, b_ref, w_ref, g2_ref):
    di = dinv_ref[...]
    s = p_ref[0] + p_ref[1] + g1_ref[...]
    z = jnp.maximum(di * s + b_ref[...], 0.0)
    g2_ref[...] = di * jnp.dot(z, w_ref[...], preferred_element_type=jnp.float32)


def _fin_body(p_ref, g2_ref, dinv_ref, b_ref, out_ref):
    s = p_ref[0] + p_ref[1] + g2_ref[...]
    out_ref[...] = dinv_ref[...] * s + b_ref[...]


def _prep_call(hist, x_pad, W1):
    return pl.pallas_call(
        _prep_body,
        grid=(GRID,),
        in_specs=[
            pl.BlockSpec((NC, R, DW), lambda i: (0, i, 0)),
            pl.BlockSpec((R, D), lambda i: (i, 0)),
            pl.BlockSpec((D, D), lambda i: (0, 0)),
        ],
        out_specs=[
            pl.BlockSpec((R, D), lambda i: (i, 0)),
            pl.BlockSpec((R, 1), lambda i: (i, 0)),
        ],
        out_shape=[
            jax.ShapeDtypeStruct((NP, D), jnp.float32),
            jax.ShapeDtypeStruct((NP, 1), jnp.float32),
        ],
    )(hist, x_pad, W1)


def _mid_call(p, g1, dinv, b1, W2):
    return pl.pallas_call(
        _mid_body,
        grid=(GRID,),
        in_specs=[
            pl.BlockSpec((NC, R, D), lambda i: (0, i, 0)),
            pl.BlockSpec((R, D), lambda i: (i, 0)),
            pl.BlockSpec((R, 1), lambda i: (i, 0)),
            pl.BlockSpec((1, D), lambda i: (0, 0)),
            pl.BlockSpec((D, D), lambda i: (0, 0)),
        ],
        out_specs=pl.BlockSpec((R, D), lambda i: (i, 0)),
        out_shape=jax.ShapeDtypeStruct((NP, D), jnp.float32),
    )(p, g1, dinv, b1, W2)


def _fin_call(p, g2, dinv, b2):
    return pl.pallas_call(
        _fin_body,
        grid=(GRID,),
        in_specs=[
            pl.BlockSpec((NC, R, D), lambda i: (0, i, 0)),
            pl.BlockSpec((R, D), lambda i: (i, 0)),
            pl.BlockSpec((R, 1), lambda i: (i, 0)),
            pl.BlockSpec((1, D), lambda i: (0, 0)),
        ],
        out_specs=pl.BlockSpec((R, D), lambda i: (i, 0)),
        out_shape=jax.ShapeDtypeStruct((NP, D), jnp.float32),
    )(p, g2, dinv, b2)


def kernel(x, edge_index, W1, b1, W2, b2):
    src = edge_index[0].astype(jnp.int32)
    dst = edge_index[1].astype(jnp.int32)

    pad = EP - E
    srcp = jnp.concatenate([src, jnp.zeros((pad,), jnp.int32)])
    dstp = jnp.concatenate([dst, jnp.full((pad,), N, jnp.int32)])

    x_pad = jnp.zeros((NP, D), jnp.float32).at[:N].set(x)

    deg_k, agg_k = _sc_kernels()
    hist = deg_k(dstp)
    g1, dinv = _prep_call(hist, x_pad, W1)
    p1 = agg_k(g1, srcp, dstp)
    g2 = _mid_call(p1, g1, dinv, b1.reshape(1, D), W2)
    p2 = agg_k(g2, srcp, dstp)
    out = _fin_call(p2, g2, dinv, b2.reshape(1, D))
    return out[:N]
